# Initial kernel scaffold; baseline (speedup 1.0000x reference)
import jax
import jax.numpy as jnp
from jax.experimental import pallas as pl


def kernel(emb, W0, a_src0, a_dst0, W1, a_src1, a_dst1, gamma, beta, entity_ids, edge_index):
    raise NotImplementedError("write your pallas kernel here")


# speedup vs baseline: 5.7055x; 5.7055x over previous
import functools

import jax
import jax.numpy as jnp
from jax.experimental import pallas as pl
from jax.experimental.pallas import tpu as pltpu

N_NODES = 10000
D = 256
H = 8
O = 32
ROW_BLK = 1000


def _dense_body(x_ref, wf_ref, bp_ref, h_ref, p_ref):
    x = x_ref[...]
    h_ref[...] = jnp.dot(x, wf_ref[...], preferred_element_type=jnp.float32)
    p_ref[...] = jnp.dot(x, bp_ref[...], preferred_element_type=jnp.float32)


def _dense(x, wf, bp):
    n = x.shape[0]
    grid = (n // ROW_BLK,)
    return pl.pallas_call(
        _dense_body,
        grid=grid,
        in_specs=[
            pl.BlockSpec((ROW_BLK, D), lambda i: (i, 0)),
            pl.BlockSpec((D, D), lambda i: (0, 0)),
            pl.BlockSpec((D, 32), lambda i: (0, 0)),
        ],
        out_specs=[
            pl.BlockSpec((ROW_BLK, D), lambda i: (i, 0)),
            pl.BlockSpec((ROW_BLK, 32), lambda i: (i, 0)),
        ],
        out_shape=[
            jax.ShapeDtypeStruct((n, D), jnp.float32),
            jax.ShapeDtypeStruct((n, 32), jnp.float32),
        ],
    )(x, wf, bp)


def _post_dense_body(agg_ref, g_ref, b_ref, wf_ref, bp_ref, h_ref, p_ref):
    x = jnp.maximum(agg_ref[...], 0.0)
    mu = jnp.mean(x, axis=-1, keepdims=True)
    var = jnp.mean((x - mu) ** 2, axis=-1, keepdims=True)
    y = (x - mu) / jnp.sqrt(var + 1e-5) * g_ref[...] + b_ref[...]
    h_ref[...] = jnp.dot(y, wf_ref[...], preferred_element_type=jnp.float32)
    p_ref[...] = jnp.dot(y, bp_ref[...], preferred_element_type=jnp.float32)


def _post_dense(agg, gamma, beta, wf, bp):
    n = agg.shape[0]
    grid = (n // ROW_BLK,)
    return pl.pallas_call(
        _post_dense_body,
        grid=grid,
        in_specs=[
            pl.BlockSpec((ROW_BLK, D), lambda i: (i, 0)),
            pl.BlockSpec((1, D), lambda i: (0, 0)),
            pl.BlockSpec((1, D), lambda i: (0, 0)),
            pl.BlockSpec((D, D), lambda i: (0, 0)),
            pl.BlockSpec((D, 32), lambda i: (0, 0)),
        ],
        out_specs=[
            pl.BlockSpec((ROW_BLK, D), lambda i: (i, 0)),
            pl.BlockSpec((ROW_BLK, 32), lambda i: (i, 0)),
        ],
        out_shape=[
            jax.ShapeDtypeStruct((n, D), jnp.float32),
            jax.ShapeDtypeStruct((n, 32), jnp.float32),
        ],
    )(agg, gamma.reshape(1, D), beta.reshape(1, D), wf, bp)


def _post_final_body(agg_ref, g_ref, b_ref, y_ref):
    x = jnp.maximum(agg_ref[...], 0.0)
    mu = jnp.mean(x, axis=-1, keepdims=True)
    var = jnp.mean((x - mu) ** 2, axis=-1, keepdims=True)
    y_ref[...] = (x - mu) / jnp.sqrt(var + 1e-5) * g_ref[...] + b_ref[...]


def _post_final(agg, gamma, beta):
    n = agg.shape[0]
    grid = (n // ROW_BLK,)
    return pl.pallas_call(
        _post_final_body,
        grid=grid,
        in_specs=[
            pl.BlockSpec((ROW_BLK, D), lambda i: (i, 0)),
            pl.BlockSpec((1, D), lambda i: (0, 0)),
            pl.BlockSpec((1, D), lambda i: (0, 0)),
        ],
        out_specs=pl.BlockSpec((ROW_BLK, D), lambda i: (i, 0)),
        out_shape=jax.ShapeDtypeStruct((n, D), jnp.float32),
    )(agg, gamma.reshape(1, D), beta.reshape(1, D))


def _leaky(x):
    return jnp.where(x > 0, x, 0.2 * x)


def _edge_softmax_agg(h_flat, p, src, dst):
    n = h_flat.shape[0]
    s_e = p[:, :8][src]
    d_e = p[:, 8:16][dst]
    e = _leaky(s_e + d_e)
    m = jax.ops.segment_max(e, dst, num_segments=n)
    m = jnp.where(jnp.isfinite(m), m, 0.0)
    pexp = jnp.exp(e - m[dst])
    denom = jax.ops.segment_sum(pexp, dst, num_segments=n)
    alpha = pexp / (denom[dst] + 1e-8)
    msg = h_flat[src].reshape(-1, H, O) * alpha[:, :, None]
    return jax.ops.segment_sum(msg.reshape(-1, H * O), dst, num_segments=n)


def _prep_weights(W, a_src, a_dst):
    wf = jnp.transpose(W, (1, 0, 2)).reshape(D, H * O)
    bs = jnp.einsum('hio,ho->ih', W, a_src[..., 0])
    bd = jnp.einsum('hio,ho->ih', W, a_dst[..., 0])
    bp = jnp.concatenate([bs, bd, bd, bs], axis=1)
    return wf, bp


def kernel(emb, W0, a_src0, a_dst0, W1, a_src1, a_dst1, gamma, beta, entity_ids, edge_index):
    src = edge_index[0]
    dst = edge_index[1]
    wf0, bp0 = _prep_weights(W0, a_src0, a_dst0)
    wf1, bp1 = _prep_weights(W1, a_src1, a_dst1)

    h1, p1 = _dense(emb, wf0, bp0)
    agg1 = _edge_softmax_agg(h1, p1, src, dst)
    h2, p2 = _post_dense(agg1, gamma, beta, wf1, bp1)
    agg2 = _edge_softmax_agg(h2, p2, src, dst)
    y = _post_final(agg2, gamma, beta)
    return y[entity_ids]
